# baseline (speedup 1.0000x reference)
import functools

import jax
import jax.numpy as jnp
from jax import lax
from jax.experimental import pallas as pl
from jax.experimental.pallas import tpu as pltpu
from jax.experimental.pallas import tpu_sc as plsc

BATCH = 16384
INPUT_DIM = 1024
CDIM = 128
NLM = 1024
TOPK = 38

RB = 4096
NB = BATCH // RB

_DEFAULT = jax.lax.Precision.DEFAULT
_HI = jax.lax.Precision.HIGHEST


def _tc_body(x_ref, w_ref, mw_ref, lm_ref, idx_ref,
             comp_ref, absum_ref, mask_ref, mlm_ref, lmn_ref):
    p = pl.program_id(0)
    i = pl.program_id(1)

    @pl.when(p == 0)
    def _phase0():
        c = lax.dot_general(x_ref[...], w_ref[...], (((1,), (1,)), ((), ())),
                            preferred_element_type=jnp.float32,
                            precision=_DEFAULT)
        comp_ref[pl.ds(i * RB, RB), :] = c
        part = jnp.sum(jnp.abs(c), axis=0, keepdims=True)

        @pl.when(i == 0)
        def _():
            absum_ref[...] = jnp.zeros_like(absum_ref)

        absum_ref[...] += part

    @pl.when(p == 1)
    def _phase1():
        @pl.when(i == 0)
        def _():
            imp = absum_ref[...] * (1.0 / BATCH) + mw_ref[...]
            ident = (lax.broadcasted_iota(jnp.int32, (CDIM, CDIM), 0)
                     == lax.broadcasted_iota(jnp.int32, (CDIM, CDIM), 1)
                     ).astype(jnp.float32)
            imp_col = lax.dot_general(ident, imp, (((1,), (1,)), ((), ())),
                                      preferred_element_type=jnp.float32,
                                      precision=_HI)
            a = jnp.broadcast_to(imp, (CDIM, CDIM))
            b = jnp.broadcast_to(imp_col, (CDIM, CDIM))
            r_iota = lax.broadcasted_iota(jnp.int32, (CDIM, CDIM), 0)
            c_iota = lax.broadcasted_iota(jnp.int32, (CDIM, CDIM), 1)
            gt = (b > a) | ((b == a) & (r_iota < c_iota))
            rank = jnp.sum(gt.astype(jnp.float32), axis=0, keepdims=True)
            mask = (rank < float(TOPK)).astype(jnp.float32)
            mask_ref[...] = mask
            lm = lm_ref[...]
            mlm_ref[...] = lm * mask
            lmn_ref[...] = jnp.sum(lm * lm, axis=1, keepdims=True)

        c = comp_ref[pl.ds(i * RB, RB), :]
        rowsum = jnp.sum((c * c) * mask_ref[...], axis=1, keepdims=True)
        rowsum_t = lax.transpose(rowsum, (1, 0))
        cross_t = lax.dot_general(mlm_ref[...], c, (((1,), (1,)), ((), ())),
                                  preferred_element_type=jnp.float32,
                                  precision=_DEFAULT)
        d2_t = (rowsum_t + lmn_ref[...]) - 2.0 * cross_t
        m = jnp.min(d2_t, axis=0, keepdims=True)
        cand = jnp.where(d2_t == m,
                         lax.broadcasted_iota(jnp.int32, (NLM, RB), 0), NLM)
        idx_ref[...] = jnp.min(cand, axis=0)


def _tc_part(x, W, mask_weights, landmarks):
    idx = pl.pallas_call(
        _tc_body,
        grid=(2, NB),
        in_specs=[
            pl.BlockSpec((RB, INPUT_DIM),
                         lambda p, i: (lax.select(p == 0, i, NB - 1), 0)),
            pl.BlockSpec((CDIM, INPUT_DIM), lambda p, i: (0, 0)),
            pl.BlockSpec((1, CDIM), lambda p, i: (0, 0)),
            pl.BlockSpec((NLM, CDIM), lambda p, i: (0, 0)),
        ],
        out_specs=pl.BlockSpec((RB,),
                               lambda p, i: (lax.select(p == 0, 0, i),)),
        out_shape=jax.ShapeDtypeStruct((BATCH,), jnp.int32),
        scratch_shapes=[
            pltpu.VMEM((BATCH, CDIM), jnp.float32),
            pltpu.VMEM((1, CDIM), jnp.float32),
            pltpu.VMEM((1, CDIM), jnp.float32),
            pltpu.VMEM((NLM, CDIM), jnp.float32),
            pltpu.VMEM((NLM, 1), jnp.float32),
        ],
    )(x, W, mask_weights.reshape(1, CDIM), landmarks)
    return idx


def _sc_gather(landmarks, idx):
    info = plsc.get_sparse_core_info()
    nc, ns = info.num_cores, info.num_subcores
    nw = nc * ns
    b_per_w = BATCH // nw
    mesh = plsc.VectorSubcoreMesh(core_axis_name="c", subcore_axis_name="s")

    chunk = 128
    nchunks = b_per_w // chunk

    @functools.partial(
        pl.kernel, mesh=mesh,
        out_type=jax.ShapeDtypeStruct((BATCH, CDIM), jnp.float32),
        scratch_types=[
            pltpu.VMEM((b_per_w,), jnp.int32),
            pltpu.VMEM((b_per_w, CDIM), jnp.float32),
            pltpu.VMEM_SHARED((NLM, CDIM), jnp.float32),
            pltpu.SemaphoreType.DMA,
            pltpu.SemaphoreType.DMA,
        ],
    )
    def gather(table_hbm, idx_hbm, out_hbm, idx_v, rows_v, table_sp, gsem,
               ssem):
        sid = lax.axis_index("s")
        wid = sid * nc + lax.axis_index("c")
        base = wid * b_per_w
        @pl.when(sid == 0)
        def _():
            pltpu.sync_copy(table_hbm, table_sp)

        pltpu.sync_copy(idx_hbm.at[pl.ds(base, b_per_w)], idx_v)
        plsc.subcore_barrier()
        gathers = []
        for c in range(nchunks):
            gathers.append(pltpu.async_copy(
                table_sp.at[idx_v.at[pl.ds(c * chunk, chunk)]],
                rows_v.at[pl.ds(c * chunk, chunk)], gsem))
        scatters = []
        for c in range(nchunks):
            gathers[c].wait()
            scatters.append(pltpu.async_copy(
                rows_v.at[pl.ds(c * chunk, chunk)],
                out_hbm.at[pl.ds(base + c * chunk, chunk)], ssem))
        for s in scatters:
            s.wait()

    return gather(landmarks, idx)


def kernel(x, W, mask_weights, landmarks):
    idx = _tc_part(x, W, mask_weights, landmarks)
    return _sc_gather(landmarks, idx)


# speedup vs baseline: 1.0030x; 1.0030x over previous
import functools

import jax
import jax.numpy as jnp
from jax import lax
from jax.experimental import pallas as pl
from jax.experimental.pallas import tpu as pltpu
from jax.experimental.pallas import tpu_sc as plsc

BATCH = 16384
INPUT_DIM = 1024
CDIM = 128
NLM = 1024
TOPK = 38

RB = 2048
NB = BATCH // RB

_DEFAULT = jax.lax.Precision.DEFAULT
_HI = jax.lax.Precision.HIGHEST


def _tc_body(x_ref, w_ref, mw_ref, lm_ref, idx_ref,
             comp_ref, absum_ref, mask_ref, mlm_ref, lmn_ref):
    p = pl.program_id(0)
    i = pl.program_id(1)

    @pl.when(p == 0)
    def _phase0():
        c = lax.dot_general(x_ref[...], w_ref[...], (((1,), (1,)), ((), ())),
                            preferred_element_type=jnp.float32,
                            precision=_DEFAULT)
        comp_ref[pl.ds(i * RB, RB), :] = c
        part = jnp.sum(jnp.abs(c), axis=0, keepdims=True)

        @pl.when(i == 0)
        def _():
            absum_ref[...] = jnp.zeros_like(absum_ref)

        absum_ref[...] += part

    @pl.when(p == 1)
    def _phase1():
        @pl.when(i == 0)
        def _():
            imp = absum_ref[...] * (1.0 / BATCH) + mw_ref[...]
            ident = (lax.broadcasted_iota(jnp.int32, (CDIM, CDIM), 0)
                     == lax.broadcasted_iota(jnp.int32, (CDIM, CDIM), 1)
                     ).astype(jnp.float32)
            imp_col = lax.dot_general(ident, imp, (((1,), (1,)), ((), ())),
                                      preferred_element_type=jnp.float32,
                                      precision=_HI)
            a = jnp.broadcast_to(imp, (CDIM, CDIM))
            b = jnp.broadcast_to(imp_col, (CDIM, CDIM))
            r_iota = lax.broadcasted_iota(jnp.int32, (CDIM, CDIM), 0)
            c_iota = lax.broadcasted_iota(jnp.int32, (CDIM, CDIM), 1)
            gt = (b > a) | ((b == a) & (r_iota < c_iota))
            rank = jnp.sum(gt.astype(jnp.float32), axis=0, keepdims=True)
            mask = (rank < float(TOPK)).astype(jnp.float32)
            mask_ref[...] = mask
            lm = lm_ref[...]
            mlm_ref[...] = lm * mask
            lmn_ref[...] = jnp.sum(lm * lm, axis=1, keepdims=True)

        c = comp_ref[pl.ds(i * RB, RB), :]
        rowsum = jnp.sum((c * c) * mask_ref[...], axis=1, keepdims=True)
        rowsum_t = lax.transpose(rowsum, (1, 0))
        cross_t = lax.dot_general(mlm_ref[...], c, (((1,), (1,)), ((), ())),
                                  preferred_element_type=jnp.float32,
                                  precision=_DEFAULT)
        d2_t = (rowsum_t + lmn_ref[...]) - 2.0 * cross_t
        m = jnp.min(d2_t, axis=0, keepdims=True)
        cand = jnp.where(d2_t == m,
                         lax.broadcasted_iota(jnp.int32, (NLM, RB), 0), NLM)
        idx_ref[...] = jnp.min(cand, axis=0)


def _tc_part(x, W, mask_weights, landmarks):
    idx = pl.pallas_call(
        _tc_body,
        grid=(2, NB),
        in_specs=[
            pl.BlockSpec((RB, INPUT_DIM),
                         lambda p, i: (lax.select(p == 0, i, NB - 1), 0)),
            pl.BlockSpec((CDIM, INPUT_DIM), lambda p, i: (0, 0)),
            pl.BlockSpec((1, CDIM), lambda p, i: (0, 0)),
            pl.BlockSpec((NLM, CDIM), lambda p, i: (0, 0)),
        ],
        out_specs=pl.BlockSpec((RB,),
                               lambda p, i: (lax.select(p == 0, 0, i),)),
        out_shape=jax.ShapeDtypeStruct((BATCH,), jnp.int32),
        scratch_shapes=[
            pltpu.VMEM((BATCH, CDIM), jnp.float32),
            pltpu.VMEM((1, CDIM), jnp.float32),
            pltpu.VMEM((1, CDIM), jnp.float32),
            pltpu.VMEM((NLM, CDIM), jnp.float32),
            pltpu.VMEM((NLM, 1), jnp.float32),
        ],
    )(x, W, mask_weights.reshape(1, CDIM), landmarks)
    return idx


def _sc_gather(landmarks, idx):
    info = plsc.get_sparse_core_info()
    nc, ns = info.num_cores, info.num_subcores
    nw = nc * ns
    b_per_w = BATCH // nw
    mesh = plsc.VectorSubcoreMesh(core_axis_name="c", subcore_axis_name="s")

    chunk = 128
    nchunks = b_per_w // chunk

    @functools.partial(
        pl.kernel, mesh=mesh,
        out_type=jax.ShapeDtypeStruct((BATCH, CDIM), jnp.float32),
        scratch_types=[
            pltpu.VMEM((b_per_w,), jnp.int32),
            pltpu.VMEM((b_per_w, CDIM), jnp.float32),
            pltpu.VMEM_SHARED((NLM, CDIM), jnp.float32),
            pltpu.SemaphoreType.DMA,
            pltpu.SemaphoreType.DMA,
        ],
    )
    def gather(table_hbm, idx_hbm, out_hbm, idx_v, rows_v, table_sp, gsem,
               ssem):
        sid = lax.axis_index("s")
        wid = sid * nc + lax.axis_index("c")
        base = wid * b_per_w
        @pl.when(sid == 0)
        def _():
            pltpu.sync_copy(table_hbm, table_sp)

        pltpu.sync_copy(idx_hbm.at[pl.ds(base, b_per_w)], idx_v)
        plsc.subcore_barrier()
        gathers = []
        for c in range(nchunks):
            gathers.append(pltpu.async_copy(
                table_sp.at[idx_v.at[pl.ds(c * chunk, chunk)]],
                rows_v.at[pl.ds(c * chunk, chunk)], gsem))
        scatters = []
        for c in range(nchunks):
            gathers[c].wait()
            scatters.append(pltpu.async_copy(
                rows_v.at[pl.ds(c * chunk, chunk)],
                out_hbm.at[pl.ds(base + c * chunk, chunk)], ssem))
        for s in scatters:
            s.wait()

    return gather(landmarks, idx)


def kernel(x, W, mask_weights, landmarks):
    idx = _tc_part(x, W, mask_weights, landmarks)
    return _sc_gather(landmarks, idx)


# speedup vs baseline: 1.0118x; 1.0087x over previous
import functools

import jax
import jax.numpy as jnp
from jax import lax
from jax.experimental import pallas as pl
from jax.experimental.pallas import tpu as pltpu
from jax.experimental.pallas import tpu_sc as plsc

BATCH = 16384
INPUT_DIM = 1024
CDIM = 128
NLM = 1024
TOPK = 38

RB = 2048
NB = BATCH // RB

_DEFAULT = jax.lax.Precision.DEFAULT
_HI = jax.lax.Precision.HIGHEST


def _tc_body(x_ref, w_ref, mw_ref, lm_ref, idx_ref,
             comp_ref, absum_ref, mask_ref, mlm_ref, lmn_ref):
    p = pl.program_id(0)
    i = pl.program_id(1)

    @pl.when(p == 0)
    def _phase0():
        c = lax.dot_general(x_ref[...], w_ref[...], (((1,), (1,)), ((), ())),
                            preferred_element_type=jnp.float32,
                            precision=_DEFAULT)
        comp_ref[pl.ds(i * RB, RB), :] = c
        part = jnp.sum(jnp.abs(c), axis=0, keepdims=True)

        @pl.when(i == 0)
        def _():
            absum_ref[...] = jnp.zeros_like(absum_ref)

        absum_ref[...] += part

    @pl.when(p == 1)
    def _phase1():
        @pl.when(i == 0)
        def _():
            imp = absum_ref[...] * (1.0 / BATCH) + mw_ref[...]
            ident = (lax.broadcasted_iota(jnp.int32, (CDIM, CDIM), 0)
                     == lax.broadcasted_iota(jnp.int32, (CDIM, CDIM), 1)
                     ).astype(jnp.float32)
            imp_col = lax.dot_general(ident, imp, (((1,), (1,)), ((), ())),
                                      preferred_element_type=jnp.float32,
                                      precision=_HI)
            a = jnp.broadcast_to(imp, (CDIM, CDIM))
            b = jnp.broadcast_to(imp_col, (CDIM, CDIM))
            r_iota = lax.broadcasted_iota(jnp.int32, (CDIM, CDIM), 0)
            c_iota = lax.broadcasted_iota(jnp.int32, (CDIM, CDIM), 1)
            gt = (b > a) | ((b == a) & (r_iota < c_iota))
            rank = jnp.sum(gt.astype(jnp.float32), axis=0, keepdims=True)
            mask = (rank < float(TOPK)).astype(jnp.float32)
            mask_ref[...] = mask
            lm = lm_ref[...]
            mlm_ref[...] = lm * mask
            lmn_ref[...] = jnp.sum(lm * lm, axis=1, keepdims=True)

        c = comp_ref[pl.ds(i * RB, RB), :]
        rowsum = jnp.sum((c * c) * mask_ref[...], axis=1, keepdims=True)
        rowsum_t = lax.transpose(rowsum, (1, 0))
        cross_t = lax.dot_general(mlm_ref[...], c, (((1,), (1,)), ((), ())),
                                  preferred_element_type=jnp.float32,
                                  precision=_DEFAULT)
        d2_t = (rowsum_t + lmn_ref[...]) - 2.0 * cross_t
        m = jnp.min(d2_t, axis=0, keepdims=True)
        cand = jnp.where(d2_t == m,
                         lax.broadcasted_iota(jnp.int32, (NLM, RB), 0), NLM)
        idx_ref[...] = jnp.min(cand, axis=0)


def _tc_part(x, W, mask_weights, landmarks):
    idx = pl.pallas_call(
        _tc_body,
        grid=(2, NB),
        in_specs=[
            pl.BlockSpec((RB, INPUT_DIM),
                         lambda p, i: (lax.select(p == 0, i, NB - 1), 0)),
            pl.BlockSpec((CDIM, INPUT_DIM), lambda p, i: (0, 0)),
            pl.BlockSpec((1, CDIM), lambda p, i: (0, 0)),
            pl.BlockSpec((NLM, CDIM), lambda p, i: (0, 0)),
        ],
        out_specs=pl.BlockSpec((RB,),
                               lambda p, i: (lax.select(p == 0, 0, i),)),
        out_shape=jax.ShapeDtypeStruct((BATCH,), jnp.int32),
        scratch_shapes=[
            pltpu.VMEM((BATCH, CDIM), jnp.float32),
            pltpu.VMEM((1, CDIM), jnp.float32),
            pltpu.VMEM((1, CDIM), jnp.float32),
            pltpu.VMEM((NLM, CDIM), jnp.float32),
            pltpu.VMEM((NLM, 1), jnp.float32),
        ],
    )(x, W, mask_weights.reshape(1, CDIM), landmarks)
    return idx


def _sc_gather(landmarks, idx):
    info = plsc.get_sparse_core_info()
    nc, ns = info.num_cores, info.num_subcores
    nw = nc * ns
    b_per_w = BATCH // nw
    mesh = plsc.VectorSubcoreMesh(core_axis_name="c", subcore_axis_name="s")

    chunk = 128
    nchunks = b_per_w // chunk

    @functools.partial(
        pl.kernel, mesh=mesh,
        out_type=jax.ShapeDtypeStruct((BATCH, CDIM), jnp.float32),
        scratch_types=[
            pltpu.VMEM((b_per_w,), jnp.int32),
            pltpu.VMEM((b_per_w, CDIM), jnp.float32),
            pltpu.VMEM_SHARED((NLM, CDIM), jnp.float32),
            pltpu.SemaphoreType.DMA,
            pltpu.SemaphoreType.DMA,
        ],
    )
    def gather(table_hbm, idx_hbm, out_hbm, idx_v, rows_v, table_sp, gsem,
               ssem):
        sid = lax.axis_index("s")
        wid = sid * nc + lax.axis_index("c")
        base = wid * b_per_w
        @pl.when(sid == 0)
        def _():
            stage = pltpu.async_copy(table_hbm, table_sp, gsem)
            pltpu.sync_copy(idx_hbm.at[pl.ds(base, b_per_w)], idx_v)
            stage.wait()

        @pl.when(sid != 0)
        def _():
            pltpu.sync_copy(idx_hbm.at[pl.ds(base, b_per_w)], idx_v)

        plsc.subcore_barrier()
        gathers = []
        for c in range(nchunks):
            gathers.append(pltpu.async_copy(
                table_sp.at[idx_v.at[pl.ds(c * chunk, chunk)]],
                rows_v.at[pl.ds(c * chunk, chunk)], gsem))
        scatters = []
        for c in range(nchunks):
            gathers[c].wait()
            scatters.append(pltpu.async_copy(
                rows_v.at[pl.ds(c * chunk, chunk)],
                out_hbm.at[pl.ds(base + c * chunk, chunk)], ssem))
        for s in scatters:
            s.wait()

    return gather(landmarks, idx)


def kernel(x, W, mask_weights, landmarks):
    idx = _tc_part(x, W, mask_weights, landmarks)
    return _sc_gather(landmarks, idx)
